# baseline (speedup 1.0000x reference)
import functools

import jax
import jax.numpy as jnp
from jax import lax
from jax.experimental import pallas as pl
from jax.experimental.pallas import tpu as pltpu
from jax.experimental.pallas import tpu_sc as plsc

N = 10000
E = 320000
D = 128
DE = 16
NW = 32
EP = 327680
ET = EP // NW
NCH = 5
CH = ET // NCH
NIV = 16
EP8 = EP * DE // 128
E8 = E * DE // 128
NP = 10112
NPS = NP // 16

_mesh = plsc.VectorSubcoreMesh(core_axis_name="c", subcore_axis_name="s")


GNCH = 8
GCH = ET // GNCH
GNIV = GCH // 128


@functools.partial(
    pl.kernel,
    mesh=_mesh,
    out_type=jax.ShapeDtypeStruct((EP, DE), jnp.float32),
    compiler_params=pltpu.CompilerParams(use_tc_tiling_on_sc=False),
    scratch_types=[
        pltpu.VMEM((ET // 128, 128), jnp.int32),
        pltpu.VMEM((GCH, DE), jnp.float32),
        pltpu.VMEM((GCH, DE), jnp.float32),
        pltpu.VMEM_SHARED((NP, DE), jnp.float32),
        pltpu.SemaphoreType.DMA,
        pltpu.SemaphoreType.DMA,
        pltpu.SemaphoreType.DMA,
        pltpu.SemaphoreType.DMA,
    ],
)
def _sc_gather(p_hbm, idx_hbm, out_hbm, idx_v, rows_a, rows_b, p_sh,
               gsem_a, gsem_b, osem_a, osem_b):
    cid = lax.axis_index("c")
    sid = lax.axis_index("s")
    wid = sid * 2 + cid

    idx_load = pltpu.async_copy(
        idx_hbm.at[pl.ds(wid * (ET // 128), ET // 128)], idx_v, gsem_a)

    @pl.when(sid == 0)
    def _():
        pltpu.sync_copy(p_hbm, p_sh.at[pl.ds(0, N)])

    idx_load.wait()
    plsc.subcore_barrier()

    def superstep(s, carry):
        for par, rows_v, gsem, osem in ((0, rows_a, gsem_a, osem_a),
                                        (1, rows_b, gsem_b, osem_b)):
            c = s * 2 + par

            @pl.when(s > 0)
            def _():
                pltpu.make_async_copy(
                    rows_v, out_hbm.at[pl.ds(0, GCH)], osem).wait()

            descs = [
                pltpu.async_copy(p_sh.at[idx_v.at[c * GNIV + j]],
                                 rows_v.at[pl.ds(j * 128, 128)], gsem)
                for j in range(GNIV)
            ]
            for d in descs:
                d.wait()
            pltpu.async_copy(rows_v,
                             out_hbm.at[pl.ds(wid * ET + c * GCH, GCH)], osem)
        return carry

    lax.fori_loop(0, GNCH // 2, superstep, 0)
    pltpu.make_async_copy(rows_a, out_hbm.at[pl.ds(0, GCH)], osem_a).wait()
    pltpu.make_async_copy(rows_b, out_hbm.at[pl.ds(0, GCH)], osem_b).wait()


@functools.partial(
    pl.kernel,
    mesh=_mesh,
    out_type=jax.ShapeDtypeStruct((2 * NP, DE), jnp.float32),
    compiler_params=pltpu.CompilerParams(use_tc_tiling_on_sc=False),
    scratch_types=[
        pltpu.VMEM((ET // 128, 128), jnp.int32),
        pltpu.VMEM((GCH, DE), jnp.float32),
        pltpu.VMEM((GCH, DE), jnp.float32),
        pltpu.VMEM((NPS, DE), jnp.float32),
        pltpu.VMEM_SHARED((NP, DE), jnp.float32),
        pltpu.SemaphoreType.DMA,
        pltpu.SemaphoreType.DMA,
        pltpu.SemaphoreType.DMA,
    ],
)
def _sc_scatter(msg_hbm, idx_hbm, out_hbm, idx_v, msg_a, msg_b, tmp_v, acc_sh,
                lsem_a, lsem_b, ssem):
    cid = lax.axis_index("c")
    sid = lax.axis_index("s")
    wid = sid * 2 + cid

    idxd = pltpu.async_copy(
        idx_hbm.at[pl.ds(wid * (ET // 128), ET // 128)], idx_v, ssem)

    def zrow(i, carry):
        tmp_v[i, :] = jnp.zeros((DE,), jnp.float32)
        return carry

    lax.fori_loop(0, NPS, zrow, 0)
    pltpu.sync_copy(tmp_v, acc_sh.at[pl.ds(sid * NPS, NPS)])
    idxd.wait()
    plsc.subcore_barrier()

    pltpu.async_copy(msg_hbm.at[pl.ds(wid * ET, GCH)], msg_a, lsem_a)

    def superstep(s, carry):
        c0 = s * 2
        pltpu.make_async_copy(msg_hbm.at[pl.ds(0, GCH)], msg_a, lsem_a).wait()

        pltpu.async_copy(msg_hbm.at[pl.ds(wid * ET + (c0 + 1) * GCH, GCH)],
                         msg_b, lsem_b)
        descs0 = [
            pltpu.async_copy(msg_a.at[pl.ds(j * 128, 128)],
                             acc_sh.at[idx_v.at[c0 * GNIV + j]],
                             ssem, add=True)
            for j in range(GNIV)
        ]
        pltpu.make_async_copy(msg_hbm.at[pl.ds(0, GCH)], msg_b, lsem_b).wait()
        for d in descs0:
            d.wait()

        @pl.when(c0 + 2 < GNCH)
        def _():
            pltpu.async_copy(
                msg_hbm.at[pl.ds(wid * ET + (c0 + 2) * GCH, GCH)],
                msg_a, lsem_a)

        descs1 = [
            pltpu.async_copy(msg_b.at[pl.ds(j * 128, 128)],
                             acc_sh.at[idx_v.at[(c0 + 1) * GNIV + j]],
                             ssem, add=True)
            for j in range(GNIV)
        ]
        for d in descs1:
            d.wait()
        return carry

    lax.fori_loop(0, GNCH // 2, superstep, 0)
    plsc.subcore_barrier()
    pltpu.sync_copy(acc_sh.at[pl.ds(sid * NPS, NPS)], tmp_v)
    pltpu.sync_copy(tmp_v, out_hbm.at[pl.ds(cid * NP + sid * NPS, NPS)])


def _p_body(x_ref, w_ref, o_ref):
    o_ref[...] = jnp.dot(x_ref[...], w_ref[...],
                         preferred_element_type=jnp.float32)


def _tc_project(x, w_xe):
    return pl.pallas_call(
        _p_body,
        out_shape=jax.ShapeDtypeStruct((N, DE), jnp.float32),
    )(x, w_xe)


BE = 2000
BQ = 8 * BE


def _t2p(eat, i16):
    del i16
    pieces = []
    for a in range(8):
        sl = eat[:, a * BE:(a + 1) * BE]
        pieces.append(jnp.transpose(sl))
    return jnp.concatenate(pieces, axis=1)


def _p2t(m8, i16):
    pieces = []
    for a in range(8):
        sl = m8[:, DE * a:DE * (a + 1)]
        pieces.append(lax.dot_general(
            i16, sl, (((1,), (1,)), ((), ())),
            preferred_element_type=jnp.float32))
    return jnp.concatenate(pieces, axis=1)


def _t2p_body(eat_ref, o_ref):
    o_ref[...] = _t2p(eat_ref[...], None)


def _tc_t2p(eat):
    return pl.pallas_call(
        _t2p_body,
        grid=(E8 // BE,),
        in_specs=[pl.BlockSpec((DE, BQ), lambda i: (0, i))],
        out_specs=pl.BlockSpec((BE, 128), lambda i: (i, 0)),
        out_shape=jax.ShapeDtypeStruct((E8, 128), jnp.float32),
    )(eat)


def _edge0_body(g_ref, ea_ref, wd_ref, be_ref, msg_ref, eao_ref):
    q = jnp.dot(ea_ref[...], wd_ref[...],
                preferred_element_type=jnp.float32) + be_ref[...]
    m = jnp.maximum(g_ref[...] + q, 0.0)
    msg_ref[...] = m
    eao_ref[...] = ea_ref[...] + m


def _tc_edge0(g8, ea8, wd, be8):
    return pl.pallas_call(
        _edge0_body,
        grid=(E8 // BE,),
        in_specs=[
            pl.BlockSpec((BE, 128), lambda i: (i, 0)),
            pl.BlockSpec((BE, 128), lambda i: (i, 0)),
            pl.BlockSpec((128, 128), lambda i: (0, 0)),
            pl.BlockSpec((1, 128), lambda i: (0, 0)),
        ],
        out_specs=[
            pl.BlockSpec((BE, 128), lambda i: (i, 0)),
            pl.BlockSpec((BE, 128), lambda i: (i, 0)),
        ],
        out_shape=[
            jax.ShapeDtypeStruct((EP8, 128), jnp.float32),
            jax.ShapeDtypeStruct((E8, 128), jnp.float32),
        ],
    )(g8, ea8, wd, be8)


def _edge1_body(g_ref, ea_ref, wd_ref, be_ref, i16_ref, msg_ref, eat_ref):
    q = jnp.dot(ea_ref[...], wd_ref[...],
                preferred_element_type=jnp.float32) + be_ref[...]
    m = jnp.maximum(g_ref[...] + q, 0.0)
    msg_ref[...] = m
    eat_ref[...] = _p2t(ea_ref[...] + m, i16_ref[...])


def _tc_edge1(g8, ea8, wd, be8, i16):
    return pl.pallas_call(
        _edge1_body,
        grid=(E8 // BE,),
        in_specs=[
            pl.BlockSpec((BE, 128), lambda i: (i, 0)),
            pl.BlockSpec((BE, 128), lambda i: (i, 0)),
            pl.BlockSpec((128, 128), lambda i: (0, 0)),
            pl.BlockSpec((1, 128), lambda i: (0, 0)),
            pl.BlockSpec((DE, DE), lambda i: (0, 0)),
        ],
        out_specs=[
            pl.BlockSpec((BE, 128), lambda i: (i, 0)),
            pl.BlockSpec((DE, BQ), lambda i: (0, i)),
        ],
        out_shape=[
            jax.ShapeDtypeStruct((EP8, 128), jnp.float32),
            jax.ShapeDtypeStruct((DE, E), jnp.float32),
        ],
    )(g8, ea8, wd, be8, i16)


def _node_body(x_ref, a0_ref, a1_ref, wnx_ref, wna_ref, bn_ref, wxe_ref,
               xo_ref, po_ref):
    agg = a0_ref[...] + a1_ref[...]
    h = jnp.dot(x_ref[...], wnx_ref[...], preferred_element_type=jnp.float32)
    h = h + jnp.dot(agg, wna_ref[...], preferred_element_type=jnp.float32)
    h = h + bn_ref[...]
    xn = x_ref[...] + jnp.maximum(h, 0.0)
    xo_ref[...] = xn
    po_ref[...] = jnp.dot(xn, wxe_ref[...], preferred_element_type=jnp.float32)


def _tc_node(x, a0, a1, w_nx, w_na, bn, w_xe):
    BN = 2000
    return pl.pallas_call(
        _node_body,
        grid=(N // BN,),
        in_specs=[
            pl.BlockSpec((BN, 128), lambda i: (i, 0)),
            pl.BlockSpec((BN, DE), lambda i: (i, 0)),
            pl.BlockSpec((BN, DE), lambda i: (i, 0)),
            pl.BlockSpec((128, 128), lambda i: (0, 0)),
            pl.BlockSpec((DE, 128), lambda i: (0, 0)),
            pl.BlockSpec((1, 128), lambda i: (0, 0)),
            pl.BlockSpec((128, DE), lambda i: (0, 0)),
        ],
        out_specs=[
            pl.BlockSpec((BN, 128), lambda i: (i, 0)),
            pl.BlockSpec((BN, DE), lambda i: (i, 0)),
        ],
        out_shape=[
            jax.ShapeDtypeStruct((N, 128), jnp.float32),
            jax.ShapeDtypeStruct((N, DE), jnp.float32),
        ],
    )(x, a0, a1, w_nx, w_na, bn, w_xe)


def kernel(x, edge_index, edge_attr, W_e, b_e, W_n, b_n):
    ei = edge_index.astype(jnp.int32)

    def _perm16(v):
        return v.reshape(E // BQ, 8, BE).transpose(0, 2, 1).reshape(E)

    row_p = jnp.concatenate(
        [_perm16(ei[0]), jnp.full((EP - E,), N, jnp.int32)]).reshape(EP // 128, 128)
    dst_p = jnp.concatenate(
        [_perm16(ei[1]), jnp.full((EP - E,), N, jnp.int32)]).reshape(EP // 128, 128)
    eat = edge_attr.T

    W_xe = W_e[:D] + W_e[D:2 * D]
    W_ee = W_e[2 * D:]
    Wd = jnp.kron(jnp.eye(8, dtype=jnp.float32), W_ee)
    be8 = jnp.tile(b_e, 8).reshape(1, 128)
    W_nx = W_n[:D]
    W_na = W_n[D:]
    bn = b_n.reshape(1, 128)

    i16 = jnp.eye(DE, dtype=jnp.float32)
    p = _tc_project(x, W_xe)
    ea8 = _tc_t2p(eat)

    g = _sc_gather(p, row_p)
    msg8, ea8 = _tc_edge0(g.reshape(EP8, 128), ea8, Wd, be8)
    aggf = _sc_scatter(msg8.reshape(EP, DE), dst_p)
    x, p = _tc_node(x, aggf[:NP], aggf[NP:], W_nx, W_na, bn, W_xe)

    g = _sc_gather(p, row_p)
    msg8, eat_out = _tc_edge1(g.reshape(EP8, 128), ea8, Wd, be8, i16)
    aggf = _sc_scatter(msg8.reshape(EP, DE), dst_p)
    x, _ = _tc_node(x, aggf[:NP], aggf[NP:], W_nx, W_na, bn, W_xe)

    return (x, eat_out.T)


# speedup vs baseline: 1.0742x; 1.0742x over previous
import functools

import jax
import jax.numpy as jnp
from jax import lax
from jax.experimental import pallas as pl
from jax.experimental.pallas import tpu as pltpu
from jax.experimental.pallas import tpu_sc as plsc

N = 10000
E = 320000
D = 128
DE = 16
NW = 32
EP = 327680
ET = EP // NW
NCH = 5
CH = ET // NCH
NIV = 16
EP8 = EP * DE // 128
E8 = E * DE // 128
NP = 10112
NPS = NP // 16

_mesh = plsc.VectorSubcoreMesh(core_axis_name="c", subcore_axis_name="s")


GNCH = 8
GCH = ET // GNCH
GNIV = GCH // 128


@functools.partial(
    pl.kernel,
    mesh=_mesh,
    out_type=jax.ShapeDtypeStruct((EP, DE), jnp.float32),
    compiler_params=pltpu.CompilerParams(use_tc_tiling_on_sc=False),
    scratch_types=[
        pltpu.VMEM((ET // 128, 128), jnp.int32),
        pltpu.VMEM((GCH, DE), jnp.float32),
        pltpu.VMEM((GCH, DE), jnp.float32),
        pltpu.VMEM_SHARED((N, DE), jnp.float32),
        pltpu.SemaphoreType.DMA,
        pltpu.SemaphoreType.DMA,
        pltpu.SemaphoreType.DMA,
        pltpu.SemaphoreType.DMA,
    ],
)
def _sc_gather(p_hbm, idx_hbm, out_hbm, idx_v, rows_a, rows_b, p_sh,
               gsem_a, gsem_b, osem_a, osem_b):
    cid = lax.axis_index("c")
    sid = lax.axis_index("s")
    wid = sid * 2 + cid

    idx_load = pltpu.async_copy(
        idx_hbm.at[pl.ds(wid * (ET // 128), ET // 128)], idx_v, gsem_a)

    @pl.when(sid == 0)
    def _():
        pltpu.sync_copy(p_hbm, p_sh)

    idx_load.wait()
    plsc.subcore_barrier()

    def superstep(s, carry):
        for par, rows_v, gsem, osem in ((0, rows_a, gsem_a, osem_a),
                                        (1, rows_b, gsem_b, osem_b)):
            c = s * 2 + par

            @pl.when(s > 0)
            def _():
                pltpu.make_async_copy(
                    rows_v, out_hbm.at[pl.ds(0, GCH)], osem).wait()

            descs = [
                pltpu.async_copy(p_sh.at[idx_v.at[c * GNIV + j]],
                                 rows_v.at[pl.ds(j * 128, 128)], gsem)
                for j in range(GNIV)
            ]
            for d in descs:
                d.wait()
            pltpu.async_copy(rows_v,
                             out_hbm.at[pl.ds(wid * ET + c * GCH, GCH)], osem)
        return carry

    lax.fori_loop(0, GNCH // 2, superstep, 0)
    pltpu.make_async_copy(rows_a, out_hbm.at[pl.ds(0, GCH)], osem_a).wait()
    pltpu.make_async_copy(rows_b, out_hbm.at[pl.ds(0, GCH)], osem_b).wait()


@functools.partial(
    pl.kernel,
    mesh=_mesh,
    out_type=jax.ShapeDtypeStruct((2 * NP, DE), jnp.float32),
    compiler_params=pltpu.CompilerParams(use_tc_tiling_on_sc=False),
    scratch_types=[
        pltpu.VMEM((ET // 128, 128), jnp.int32),
        pltpu.VMEM((GCH, DE), jnp.float32),
        pltpu.VMEM((GCH, DE), jnp.float32),
        pltpu.VMEM((NPS, DE), jnp.float32),
        pltpu.VMEM_SHARED((NP, DE), jnp.float32),
        pltpu.SemaphoreType.DMA,
        pltpu.SemaphoreType.DMA,
        pltpu.SemaphoreType.DMA,
    ],
)
def _sc_scatter(msg_hbm, idx_hbm, out_hbm, idx_v, msg_a, msg_b, tmp_v, acc_sh,
                lsem_a, lsem_b, ssem):
    cid = lax.axis_index("c")
    sid = lax.axis_index("s")
    wid = sid * 2 + cid

    idxd = pltpu.async_copy(
        idx_hbm.at[pl.ds(wid * (ET // 128), ET // 128)], idx_v, ssem)

    def zrow(i, carry):
        tmp_v[i, :] = jnp.zeros((DE,), jnp.float32)
        return carry

    lax.fori_loop(0, NPS, zrow, 0)
    pltpu.sync_copy(tmp_v, acc_sh.at[pl.ds(sid * NPS, NPS)])
    idxd.wait()
    plsc.subcore_barrier()

    pltpu.async_copy(msg_hbm.at[pl.ds(wid * ET, GCH)], msg_a, lsem_a)

    def superstep(s, carry):
        c0 = s * 2
        pltpu.make_async_copy(msg_hbm.at[pl.ds(0, GCH)], msg_a, lsem_a).wait()

        pltpu.async_copy(msg_hbm.at[pl.ds(wid * ET + (c0 + 1) * GCH, GCH)],
                         msg_b, lsem_b)
        descs0 = [
            pltpu.async_copy(msg_a.at[pl.ds(j * 128, 128)],
                             acc_sh.at[idx_v.at[c0 * GNIV + j]],
                             ssem, add=True)
            for j in range(GNIV)
        ]
        pltpu.make_async_copy(msg_hbm.at[pl.ds(0, GCH)], msg_b, lsem_b).wait()
        for d in descs0:
            d.wait()

        @pl.when(c0 + 2 < GNCH)
        def _():
            pltpu.async_copy(
                msg_hbm.at[pl.ds(wid * ET + (c0 + 2) * GCH, GCH)],
                msg_a, lsem_a)

        descs1 = [
            pltpu.async_copy(msg_b.at[pl.ds(j * 128, 128)],
                             acc_sh.at[idx_v.at[(c0 + 1) * GNIV + j]],
                             ssem, add=True)
            for j in range(GNIV)
        ]
        for d in descs1:
            d.wait()
        return carry

    lax.fori_loop(0, GNCH // 2, superstep, 0)
    plsc.subcore_barrier()
    pltpu.sync_copy(acc_sh.at[pl.ds(sid * NPS, NPS)], tmp_v)
    pltpu.sync_copy(tmp_v, out_hbm.at[pl.ds(cid * NP + sid * NPS, NPS)])


def _p_body(x_ref, w_ref, o_ref):
    o_ref[...] = jnp.dot(x_ref[...], w_ref[...],
                         preferred_element_type=jnp.float32)


def _tc_project(x, w_xe):
    return pl.pallas_call(
        _p_body,
        out_shape=jax.ShapeDtypeStruct((N, DE), jnp.float32),
    )(x, w_xe)


BE = 2000
BQ = 8 * BE


def _t2p(eat, i16):
    pieces = []
    for a in range(8):
        sl = eat[:, a * BE:(a + 1) * BE]
        pieces.append(lax.dot_general(
            sl, i16, (((0,), (0,)), ((), ())),
            preferred_element_type=jnp.float32))
    return jnp.concatenate(pieces, axis=1)


def _p2t(m8, i16):
    pieces = []
    for a in range(8):
        sl = m8[:, DE * a:DE * (a + 1)]
        pieces.append(lax.dot_general(
            i16, sl, (((1,), (1,)), ((), ())),
            preferred_element_type=jnp.float32))
    return jnp.concatenate(pieces, axis=1)


def _edge0_body(g_ref, eat_ref, wd_ref, be_ref, i16_ref, msg_ref, eao_ref):
    ea8 = _t2p(eat_ref[...], i16_ref[...])
    q = jnp.dot(ea8, wd_ref[...],
                preferred_element_type=jnp.float32) + be_ref[...]
    m = jnp.maximum(g_ref[...] + q, 0.0)
    msg_ref[...] = m
    eao_ref[...] = ea8 + m


def _tc_edge0(g8, eat, wd, be8, i16):
    return pl.pallas_call(
        _edge0_body,
        grid=(E8 // BE,),
        in_specs=[
            pl.BlockSpec((BE, 128), lambda i: (i, 0)),
            pl.BlockSpec((DE, BQ), lambda i: (0, i)),
            pl.BlockSpec((128, 128), lambda i: (0, 0)),
            pl.BlockSpec((1, 128), lambda i: (0, 0)),
            pl.BlockSpec((DE, DE), lambda i: (0, 0)),
        ],
        out_specs=[
            pl.BlockSpec((BE, 128), lambda i: (i, 0)),
            pl.BlockSpec((BE, 128), lambda i: (i, 0)),
        ],
        out_shape=[
            jax.ShapeDtypeStruct((EP8, 128), jnp.float32),
            jax.ShapeDtypeStruct((E8, 128), jnp.float32),
        ],
    )(g8, eat, wd, be8, i16)


def _edge1_body(g_ref, ea_ref, wd_ref, be_ref, i16_ref, msg_ref, eat_ref):
    q = jnp.dot(ea_ref[...], wd_ref[...],
                preferred_element_type=jnp.float32) + be_ref[...]
    m = jnp.maximum(g_ref[...] + q, 0.0)
    msg_ref[...] = m
    eat_ref[...] = _p2t(ea_ref[...] + m, i16_ref[...])


def _tc_edge1(g8, ea8, wd, be8, i16):
    return pl.pallas_call(
        _edge1_body,
        grid=(E8 // BE,),
        in_specs=[
            pl.BlockSpec((BE, 128), lambda i: (i, 0)),
            pl.BlockSpec((BE, 128), lambda i: (i, 0)),
            pl.BlockSpec((128, 128), lambda i: (0, 0)),
            pl.BlockSpec((1, 128), lambda i: (0, 0)),
            pl.BlockSpec((DE, DE), lambda i: (0, 0)),
        ],
        out_specs=[
            pl.BlockSpec((BE, 128), lambda i: (i, 0)),
            pl.BlockSpec((DE, BQ), lambda i: (0, i)),
        ],
        out_shape=[
            jax.ShapeDtypeStruct((EP8, 128), jnp.float32),
            jax.ShapeDtypeStruct((DE, E), jnp.float32),
        ],
    )(g8, ea8, wd, be8, i16)


def _node_body(x_ref, a0_ref, a1_ref, wnx_ref, wna_ref, bn_ref, wxe_ref,
               xo_ref, po_ref):
    agg = a0_ref[...] + a1_ref[...]
    h = jnp.dot(x_ref[...], wnx_ref[...], preferred_element_type=jnp.float32)
    h = h + jnp.dot(agg, wna_ref[...], preferred_element_type=jnp.float32)
    h = h + bn_ref[...]
    xn = x_ref[...] + jnp.maximum(h, 0.0)
    xo_ref[...] = xn
    po_ref[...] = jnp.dot(xn, wxe_ref[...], preferred_element_type=jnp.float32)


def _tc_node(x, a0, a1, w_nx, w_na, bn, w_xe):
    BN = 2000
    return pl.pallas_call(
        _node_body,
        grid=(N // BN,),
        in_specs=[
            pl.BlockSpec((BN, 128), lambda i: (i, 0)),
            pl.BlockSpec((BN, DE), lambda i: (i, 0)),
            pl.BlockSpec((BN, DE), lambda i: (i, 0)),
            pl.BlockSpec((128, 128), lambda i: (0, 0)),
            pl.BlockSpec((DE, 128), lambda i: (0, 0)),
            pl.BlockSpec((1, 128), lambda i: (0, 0)),
            pl.BlockSpec((128, DE), lambda i: (0, 0)),
        ],
        out_specs=[
            pl.BlockSpec((BN, 128), lambda i: (i, 0)),
            pl.BlockSpec((BN, DE), lambda i: (i, 0)),
        ],
        out_shape=[
            jax.ShapeDtypeStruct((N, 128), jnp.float32),
            jax.ShapeDtypeStruct((N, DE), jnp.float32),
        ],
    )(x, a0, a1, w_nx, w_na, bn, w_xe)


def kernel(x, edge_index, edge_attr, W_e, b_e, W_n, b_n):
    ei = edge_index.astype(jnp.int32)

    def _perm16(v):
        return v.reshape(E // BQ, 8, BE).transpose(0, 2, 1).reshape(E)

    row_p = jnp.concatenate(
        [_perm16(ei[0]), jnp.zeros((EP - E,), jnp.int32)]).reshape(EP // 128, 128)
    dst_p = jnp.concatenate(
        [_perm16(ei[1]), jnp.full((EP - E,), N, jnp.int32)]).reshape(EP // 128, 128)
    eat = edge_attr.T

    W_xe = W_e[:D] + W_e[D:2 * D]
    W_ee = W_e[2 * D:]
    Wd = jnp.kron(jnp.eye(8, dtype=jnp.float32), W_ee)
    be8 = jnp.tile(b_e, 8).reshape(1, 128)
    W_nx = W_n[:D]
    W_na = W_n[D:]
    bn = b_n.reshape(1, 128)

    i16 = jnp.eye(DE, dtype=jnp.float32)
    p = _tc_project(x, W_xe)

    g = _sc_gather(p, row_p)
    msg8, ea8 = _tc_edge0(g.reshape(EP8, 128), eat, Wd, be8, i16)
    aggf = _sc_scatter(msg8.reshape(EP, DE), dst_p)
    x, p = _tc_node(x, aggf[:NP], aggf[NP:], W_nx, W_na, bn, W_xe)

    g = _sc_gather(p, row_p)
    msg8, eat_out = _tc_edge1(g.reshape(EP8, 128), ea8, Wd, be8, i16)
    aggf = _sc_scatter(msg8.reshape(EP, DE), dst_p)
    x, _ = _tc_node(x, aggf[:NP], aggf[NP:], W_nx, W_na, bn, W_xe)

    return (x, eat_out.T)
